# baseline (speedup 1.0000x reference)
import functools

import jax
import jax.numpy as jnp
from jax import lax
from jax.experimental import pallas as pl
from jax.experimental.pallas import tpu as pltpu
from jax.experimental.pallas import tpu_sc as plsc

GAMMA = 3.0
N = 625
NPAD = 640
B = 16
NCHUNK = NPAD // 16
NROWS = 14


def _sc_task_body(x_hbm, out_hbm, xv, kv, uv, vv, sv):
    cid = lax.axis_index("c")
    sid = lax.axis_index("s")

    pltpu.sync_copy(x_hbm.at[sid], xv)

    ones = jnp.full((16,), 1.0, jnp.float32)
    zeros = jnp.zeros((16,), jnp.float32)
    w1 = jnp.full((16,), 1.0 - cid.astype(jnp.float32), jnp.float32)
    w2 = ones - w1

    pacc = zeros
    for c in range(NCHUNK):
        ds = pl.ds(c * 16, 16)
        cls1 = xv[0, ds]
        labf = xv[1, ds]
        mf = jnp.where(labf > 0.5, ones, zeros)
        pp = jnp.exp(cls1)

        sh0 = xv[10, ds]
        sh1 = xv[11, ds]
        sh2 = xv[12, ds]
        sh3 = xv[13, ds]

        def corners(base):
            cx = xv[base + 0, ds] * sh2 + sh0
            cy = xv[base + 1, ds] * sh3 + sh1
            w = jnp.exp(xv[base + 2, ds]) * sh2
            h = jnp.exp(xv[base + 3, ds]) * sh3
            hw = w * 0.5
            hh = h * 0.5
            return cx - hw, cy - hh, cx + hw, cy + hh

        ax1, ay1, ax2, ay2 = corners(2)
        bx1, by1, bx2, by2 = corners(6)

        ix1 = jnp.maximum(ax1, bx1)
        iy1 = jnp.maximum(ay1, by1)
        ix2 = jnp.minimum(ax2, bx2)
        iy2 = jnp.minimum(ay2, by2)
        inter = jnp.maximum(ix2 - ix1, 0.0) * jnp.maximum(iy2 - iy1, 0.0)
        area_a = jnp.maximum(ax2 - ax1, 0.0) * jnp.maximum(ay2 - ay1, 0.0)
        area_b = jnp.maximum(bx2 - bx1, 0.0) * jnp.maximum(by2 - by1, 0.0)
        iou = inter / jnp.maximum(area_a + area_b - inter, 1e-6)

        key = w1 * iou + w2 * pp
        val = w1 * pp + w2 * iou
        kv[ds] = key
        uv[ds] = mf * jnp.exp(-GAMMA * val)
        vv[ds] = mf * jnp.exp(GAMMA * val)
        pacc = pacc + mf

    iotav = lax.broadcasted_iota(jnp.int32, (16,), 0)
    gdn = lax.GatherDimensionNumbers(
        offset_dims=(), collapsed_slice_dims=(0,), start_index_map=(0,))

    def lane_bcast(vec, j):
        idx = jnp.full((16,), j, jnp.int32)
        return lax.gather(vec, idx[:, None], gdn, (1,),
                          mode=lax.GatherScatterMode.PROMISE_IN_BOUNDS)

    def body(b, sacc):
        bvec = jnp.full((16,), b, jnp.int32)
        start = (b // 16) * 16
        j = b - start
        kb = lane_bcast(kv[pl.ds(start, 16)], j)
        vb = lane_bcast(vv[pl.ds(start, 16)], j)
        accs = [zeros, zeros, zeros, zeros]
        for c in range(NCHUNK):
            ds = pl.ds(c * 16, 16)
            ka = kv[ds]
            ua = uv[ds]
            ia = iotav + (c * 16)
            pred = (ka > kb) | ((ka == kb) & (ia < bvec))
            accs[c % 4] = accs[c % 4] + jnp.where(pred, ua, zeros)
        acc = (accs[0] + accs[1]) + (accs[2] + accs[3])
        return sacc + acc * vb

    sacc = lax.fori_loop(0, NPAD, body, zeros)

    sv[0, :] = sacc
    sv[1, :] = pacc
    pltpu.sync_copy(sv, out_hbm.at[sid * 2 + cid])


def _sc_call(x):
    mesh = plsc.VectorSubcoreMesh(core_axis_name="c", subcore_axis_name="s")
    k = functools.partial(
        pl.kernel,
        mesh=mesh,
        out_type=jax.ShapeDtypeStruct((2 * B, 2, 16), jnp.float32),
        scratch_types=[
            pltpu.VMEM((NROWS, NPAD), jnp.float32),
            pltpu.VMEM((NPAD,), jnp.float32),
            pltpu.VMEM((NPAD,), jnp.float32),
            pltpu.VMEM((NPAD,), jnp.float32),
            pltpu.VMEM((2, 16), jnp.float32),
        ],
    )(_sc_task_body)
    return k(x)


def _finalize_kernel(x_ref, f1_ref, f2_ref):
    x = x_ref[...]
    s = jnp.sum(x[:, 0, :], axis=1, keepdims=True)
    p = jnp.sum(x[:, 1, :], axis=1, keepdims=True)
    rowid = lax.broadcasted_iota(jnp.int32, (2 * B, 1), 0)
    is1 = (rowid % 2) == 0
    npairs = jnp.maximum(p * (p - 1.0) * 0.5, 1.0)
    include = (p >= 2.0).astype(jnp.float32)
    contrib = include * s / npairs
    total1 = jnp.sum(jnp.where(is1, contrib, 0.0))
    total2 = jnp.sum(jnp.where(is1, 0.0, contrib))
    count = jnp.sum(jnp.where(is1, include, 0.0))
    denom = jnp.maximum(count, 1.0)
    has = (count > 0.0).astype(jnp.float32)
    f1_ref[...] = (total1 / denom * has).reshape(1, 1)
    f2_ref[...] = (total2 / denom * has).reshape(1, 1)


def kernel(cls, label_cls, pred_loc, label_loc, shape):
    pad = NPAD - N
    cls1 = jnp.pad(cls.reshape(B, N, 2)[:, :, 1], ((0, 0), (0, pad)))
    labf = jnp.pad(label_cls.reshape(B, N).astype(jnp.float32),
                   ((0, 0), (0, pad)))
    ploc = jnp.pad(pred_loc.reshape(B, 4, N), ((0, 0), (0, 0), (0, pad)))
    lloc = jnp.pad(label_loc.reshape(B, 4, N), ((0, 0), (0, 0), (0, pad)))
    shp = jnp.pad(shape.reshape(4, N), ((0, 0), (0, pad)),
                  constant_values=1.0)
    shp_b = jnp.broadcast_to(shp[None], (B, 4, NPAD))
    x = jnp.concatenate(
        [cls1[:, None, :], labf[:, None, :], ploc, lloc, shp_b], axis=1)

    parts = _sc_call(x)

    f1, f2 = pl.pallas_call(
        _finalize_kernel,
        out_shape=[
            jax.ShapeDtypeStruct((1, 1), jnp.float32),
            jax.ShapeDtypeStruct((1, 1), jnp.float32),
        ],
    )(parts)
    return (f1.reshape(()), f2.reshape(()))


# speedup vs baseline: 8.2689x; 8.2689x over previous
import functools

import jax
import jax.numpy as jnp
from jax import lax
from jax.experimental import pallas as pl
from jax.experimental.pallas import tpu as pltpu
from jax.experimental.pallas import tpu_sc as plsc

GAMMA = 3.0
N = 625
NPAD = 640
B = 16
NCHUNK = NPAD // 16
NROWS = 14


def _sc_task_body(x_hbm, out_hbm, xv, kv, uv, vv, sv):
    cid = lax.axis_index("c")
    sid = lax.axis_index("s")

    pltpu.sync_copy(x_hbm.at[sid], xv)

    ones = jnp.full((16,), 1.0, jnp.float32)
    zeros = jnp.zeros((16,), jnp.float32)
    w1 = jnp.full((16,), 1.0 - cid.astype(jnp.float32), jnp.float32)
    w2 = ones - w1

    pacc = zeros
    for c in range(NCHUNK):
        ds = pl.ds(c * 16, 16)
        cls1 = xv[0, ds]
        labf = xv[1, ds]
        mf = jnp.where(labf > 0.5, ones, zeros)
        pp = jnp.exp(cls1)

        sh0 = xv[10, ds]
        sh1 = xv[11, ds]
        sh2 = xv[12, ds]
        sh3 = xv[13, ds]

        def corners(base):
            cx = xv[base + 0, ds] * sh2 + sh0
            cy = xv[base + 1, ds] * sh3 + sh1
            w = jnp.exp(xv[base + 2, ds]) * sh2
            h = jnp.exp(xv[base + 3, ds]) * sh3
            hw = w * 0.5
            hh = h * 0.5
            return cx - hw, cy - hh, cx + hw, cy + hh

        ax1, ay1, ax2, ay2 = corners(2)
        bx1, by1, bx2, by2 = corners(6)

        ix1 = jnp.maximum(ax1, bx1)
        iy1 = jnp.maximum(ay1, by1)
        ix2 = jnp.minimum(ax2, bx2)
        iy2 = jnp.minimum(ay2, by2)
        inter = jnp.maximum(ix2 - ix1, 0.0) * jnp.maximum(iy2 - iy1, 0.0)
        area_a = jnp.maximum(ax2 - ax1, 0.0) * jnp.maximum(ay2 - ay1, 0.0)
        area_b = jnp.maximum(bx2 - bx1, 0.0) * jnp.maximum(by2 - by1, 0.0)
        iou = inter / jnp.maximum(area_a + area_b - inter, 1e-6)

        key = w1 * iou + w2 * pp
        val = w1 * pp + w2 * iou
        kv[ds] = key
        uv[ds] = mf * jnp.exp(-GAMMA * val)
        vv[ds] = mf * jnp.exp(GAMMA * val)
        pacc = pacc + mf

    iotav = lax.broadcasted_iota(jnp.int32, (16,), 0)
    gdn = lax.GatherDimensionNumbers(
        offset_dims=(), collapsed_slice_dims=(0,), start_index_map=(0,))

    def lane_bcast(vec, j):
        idx = jnp.full((16,), j, jnp.int32)
        return lax.gather(vec, idx[:, None], gdn, (1,),
                          mode=lax.GatherScatterMode.PROMISE_IN_BOUNDS)

    def body(bc, sacc):
        base = bc * 16
        bds = pl.ds(base, 16)
        kb16 = kv[bds]
        kbs = [lane_bcast(kb16, j) for j in range(16)]
        accs0 = (zeros,) * 16

        def low(c, accs):
            ka = kv[pl.ds(c * 16, 16)]
            ua = uv[pl.ds(c * 16, 16)]
            return tuple(
                accs[j] + jnp.where(ka >= kbs[j], ua, zeros)
                for j in range(16))

        def up(c, accs):
            ka = kv[pl.ds(c * 16, 16)]
            ua = uv[pl.ds(c * 16, 16)]
            return tuple(
                accs[j] + jnp.where(ka > kbs[j], ua, zeros)
                for j in range(16))

        accs = lax.fori_loop(0, bc, low, accs0)
        accs = lax.fori_loop(bc + 1, NCHUNK, up, accs)

        udiag = uv[bds]
        accs = list(accs)
        for j in range(16):
            tie = (kb16 == kbs[j]) & (iotav < j)
            pred = (kb16 > kbs[j]) | tie
            accs[j] = accs[j] + jnp.where(pred, udiag, zeros)

        vbc = vv[bds]
        total = zeros
        for j in range(16):
            total = total + accs[j] * lane_bcast(vbc, j)
        return sacc + total

    sacc = lax.fori_loop(0, NCHUNK, body, zeros)

    sv[0, :] = sacc
    sv[1, :] = pacc
    pltpu.sync_copy(sv, out_hbm.at[sid * 2 + cid])


def _sc_call(x):
    mesh = plsc.VectorSubcoreMesh(core_axis_name="c", subcore_axis_name="s")
    k = functools.partial(
        pl.kernel,
        mesh=mesh,
        out_type=jax.ShapeDtypeStruct((2 * B, 2, 16), jnp.float32),
        scratch_types=[
            pltpu.VMEM((NROWS, NPAD), jnp.float32),
            pltpu.VMEM((NPAD,), jnp.float32),
            pltpu.VMEM((NPAD,), jnp.float32),
            pltpu.VMEM((NPAD,), jnp.float32),
            pltpu.VMEM((2, 16), jnp.float32),
        ],
    )(_sc_task_body)
    return k(x)


def _finalize_kernel(x_ref, f1_ref, f2_ref):
    x = x_ref[...]
    s = jnp.sum(x[:, 0, :], axis=1, keepdims=True)
    p = jnp.sum(x[:, 1, :], axis=1, keepdims=True)
    rowid = lax.broadcasted_iota(jnp.int32, (2 * B, 1), 0)
    is1 = (rowid % 2) == 0
    npairs = jnp.maximum(p * (p - 1.0) * 0.5, 1.0)
    include = (p >= 2.0).astype(jnp.float32)
    contrib = include * s / npairs
    total1 = jnp.sum(jnp.where(is1, contrib, 0.0))
    total2 = jnp.sum(jnp.where(is1, 0.0, contrib))
    count = jnp.sum(jnp.where(is1, include, 0.0))
    denom = jnp.maximum(count, 1.0)
    has = (count > 0.0).astype(jnp.float32)
    f1_ref[...] = (total1 / denom * has).reshape(1, 1)
    f2_ref[...] = (total2 / denom * has).reshape(1, 1)


def kernel(cls, label_cls, pred_loc, label_loc, shape):
    pad = NPAD - N
    cls1 = jnp.pad(cls.reshape(B, N, 2)[:, :, 1], ((0, 0), (0, pad)))
    labf = jnp.pad(label_cls.reshape(B, N).astype(jnp.float32),
                   ((0, 0), (0, pad)))
    ploc = jnp.pad(pred_loc.reshape(B, 4, N), ((0, 0), (0, 0), (0, pad)))
    lloc = jnp.pad(label_loc.reshape(B, 4, N), ((0, 0), (0, 0), (0, pad)))
    shp = jnp.pad(shape.reshape(4, N), ((0, 0), (0, pad)),
                  constant_values=1.0)
    shp_b = jnp.broadcast_to(shp[None], (B, 4, NPAD))
    x = jnp.concatenate(
        [cls1[:, None, :], labf[:, None, :], ploc, lloc, shp_b], axis=1)

    parts = _sc_call(x)

    f1, f2 = pl.pallas_call(
        _finalize_kernel,
        out_shape=[
            jax.ShapeDtypeStruct((1, 1), jnp.float32),
            jax.ShapeDtypeStruct((1, 1), jnp.float32),
        ],
    )(parts)
    return (f1.reshape(()), f2.reshape(()))


# speedup vs baseline: 8.8379x; 1.0688x over previous
import functools

import jax
import jax.numpy as jnp
from jax import lax
from jax.experimental import pallas as pl
from jax.experimental.pallas import tpu as pltpu
from jax.experimental.pallas import tpu_sc as plsc

GAMMA = 3.0
N = 625
NPAD = 640
B = 16
NCHUNK = NPAD // 16
NROWS = 14


def _sc_task_body(c1_hbm, lab_hbm, ploc_hbm, lloc_hbm, shp_hbm, out_hbm,
                  c1v, lv, pv, llv, shv, kv, uv, vv, sv):
    cid = lax.axis_index("c")
    sid = lax.axis_index("s")

    pltpu.sync_copy(c1_hbm.at[sid], c1v)
    pltpu.sync_copy(lab_hbm.at[sid], lv)
    pltpu.sync_copy(ploc_hbm.at[sid], pv)
    pltpu.sync_copy(lloc_hbm.at[sid], llv)
    pltpu.sync_copy(shp_hbm, shv)

    ones = jnp.full((16,), 1.0, jnp.float32)
    zeros = jnp.zeros((16,), jnp.float32)
    w1 = jnp.full((16,), 1.0 - cid.astype(jnp.float32), jnp.float32)
    w2 = ones - w1

    pacc = zeros
    for c in range(NCHUNK):
        ds = pl.ds(c * 16, 16)
        cls1 = c1v[ds]
        labf = lv[ds]
        mf = jnp.where(labf > 0.5, ones, zeros)
        pp = jnp.exp(cls1)

        sh0 = shv[0, ds]
        sh1 = shv[1, ds]
        sh2 = shv[2, ds]
        sh3 = shv[3, ds]

        def corners(locv):
            cx = locv[0, ds] * sh2 + sh0
            cy = locv[1, ds] * sh3 + sh1
            w = jnp.exp(locv[2, ds]) * sh2
            h = jnp.exp(locv[3, ds]) * sh3
            hw = w * 0.5
            hh = h * 0.5
            return cx - hw, cy - hh, cx + hw, cy + hh

        ax1, ay1, ax2, ay2 = corners(pv)
        bx1, by1, bx2, by2 = corners(llv)

        ix1 = jnp.maximum(ax1, bx1)
        iy1 = jnp.maximum(ay1, by1)
        ix2 = jnp.minimum(ax2, bx2)
        iy2 = jnp.minimum(ay2, by2)
        inter = jnp.maximum(ix2 - ix1, 0.0) * jnp.maximum(iy2 - iy1, 0.0)
        area_a = jnp.maximum(ax2 - ax1, 0.0) * jnp.maximum(ay2 - ay1, 0.0)
        area_b = jnp.maximum(bx2 - bx1, 0.0) * jnp.maximum(by2 - by1, 0.0)
        iou = inter / jnp.maximum(area_a + area_b - inter, 1e-6)

        key = w1 * iou + w2 * pp
        val = w1 * pp + w2 * iou
        kv[ds] = key
        uv[ds] = mf * jnp.exp(-GAMMA * val)
        vv[ds] = mf * jnp.exp(GAMMA * val)
        pacc = pacc + mf

    iotav = lax.broadcasted_iota(jnp.int32, (16,), 0)
    gdn = lax.GatherDimensionNumbers(
        offset_dims=(), collapsed_slice_dims=(0,), start_index_map=(0,))

    def lane_bcast(vec, j):
        idx = jnp.full((16,), j, jnp.int32)
        return lax.gather(vec, idx[:, None], gdn, (1,),
                          mode=lax.GatherScatterMode.PROMISE_IN_BOUNDS)

    def body(bc, sacc):
        base = bc * 16
        bds = pl.ds(base, 16)
        kb16 = kv[bds]
        kbs = [lane_bcast(kb16, j) for j in range(16)]
        accs0 = (zeros,) * 16

        def low(c, accs):
            ka = kv[pl.ds(c * 16, 16)]
            ua = uv[pl.ds(c * 16, 16)]
            return tuple(
                accs[j] + jnp.where(ka >= kbs[j], ua, zeros)
                for j in range(16))

        def up(c, accs):
            ka = kv[pl.ds(c * 16, 16)]
            ua = uv[pl.ds(c * 16, 16)]
            return tuple(
                accs[j] + jnp.where(ka > kbs[j], ua, zeros)
                for j in range(16))

        accs = lax.fori_loop(0, bc, low, accs0)
        accs = lax.fori_loop(bc + 1, NCHUNK, up, accs)

        udiag = uv[bds]
        accs = list(accs)
        for j in range(16):
            tie = (kb16 == kbs[j]) & (iotav < j)
            pred = (kb16 > kbs[j]) | tie
            accs[j] = accs[j] + jnp.where(pred, udiag, zeros)

        vbc = vv[bds]
        total = zeros
        for j in range(16):
            total = total + accs[j] * lane_bcast(vbc, j)
        return sacc + total

    sacc = lax.fori_loop(0, NCHUNK, body, zeros)

    sv[0, :] = sacc
    sv[1, :] = pacc
    pltpu.sync_copy(sv, out_hbm.at[sid * 2 + cid])


def _sc_call(x):
    mesh = plsc.VectorSubcoreMesh(core_axis_name="c", subcore_axis_name="s")
    k = functools.partial(
        pl.kernel,
        mesh=mesh,
        out_type=jax.ShapeDtypeStruct((2 * B, 2, 16), jnp.float32),
        scratch_types=[
            pltpu.VMEM((NPAD,), jnp.float32),
            pltpu.VMEM((NPAD,), jnp.float32),
            pltpu.VMEM((4, NPAD), jnp.float32),
            pltpu.VMEM((4, NPAD), jnp.float32),
            pltpu.VMEM((4, NPAD), jnp.float32),
            pltpu.VMEM((NPAD,), jnp.float32),
            pltpu.VMEM((NPAD,), jnp.float32),
            pltpu.VMEM((NPAD,), jnp.float32),
            pltpu.VMEM((2, 16), jnp.float32),
        ],
    )(_sc_task_body)
    return k(*x)


def _finalize_kernel(x_ref, f1_ref, f2_ref):
    x = x_ref[...]
    s = jnp.sum(x[:, 0, :], axis=1, keepdims=True)
    p = jnp.sum(x[:, 1, :], axis=1, keepdims=True)
    rowid = lax.broadcasted_iota(jnp.int32, (2 * B, 1), 0)
    is1 = (rowid % 2) == 0
    npairs = jnp.maximum(p * (p - 1.0) * 0.5, 1.0)
    include = (p >= 2.0).astype(jnp.float32)
    contrib = include * s / npairs
    total1 = jnp.sum(jnp.where(is1, contrib, 0.0))
    total2 = jnp.sum(jnp.where(is1, 0.0, contrib))
    count = jnp.sum(jnp.where(is1, include, 0.0))
    denom = jnp.maximum(count, 1.0)
    has = (count > 0.0).astype(jnp.float32)
    f1_ref[...] = (total1 / denom * has).reshape(1, 1)
    f2_ref[...] = (total2 / denom * has).reshape(1, 1)


def kernel(cls, label_cls, pred_loc, label_loc, shape):
    pad = NPAD - N
    cls1 = jnp.pad(cls.reshape(B, N, 2)[:, :, 1], ((0, 0), (0, pad)))
    labf = jnp.pad(label_cls.reshape(B, N).astype(jnp.float32),
                   ((0, 0), (0, pad)))
    ploc = jnp.pad(pred_loc.reshape(B, 4, N), ((0, 0), (0, 0), (0, pad)))
    lloc = jnp.pad(label_loc.reshape(B, 4, N), ((0, 0), (0, 0), (0, pad)))
    shp = jnp.pad(shape.reshape(4, N), ((0, 0), (0, pad)),
                  constant_values=1.0)

    parts = _sc_call((cls1, labf, ploc, lloc, shp))

    f1, f2 = pl.pallas_call(
        _finalize_kernel,
        out_shape=[
            jax.ShapeDtypeStruct((1, 1), jnp.float32),
            jax.ShapeDtypeStruct((1, 1), jnp.float32),
        ],
    )(parts)
    return (f1.reshape(()), f2.reshape(()))


# speedup vs baseline: 11.0609x; 1.2515x over previous
import functools

import jax
import jax.numpy as jnp
from jax import lax
from jax.experimental import pallas as pl
from jax.experimental.pallas import tpu as pltpu
from jax.experimental.pallas import tpu_sc as plsc

GAMMA = 3.0
N = 625
NPAD = 640
B = 16
NCHUNK = NPAD // 16
NSC = 4
NTC = B - NSC
Q = 16 // NSC
CPS = NCHUNK // Q


def _sc_task_body(c1_hbm, lab_hbm, ploc_hbm, lloc_hbm, shp_hbm, out_hbm,
                  c1v, lv, pv, llv, shv, kv, uv, vv, sv):
    cid = lax.axis_index("c")
    sid = lax.axis_index("s")
    sample = sid // Q
    bslice = sid % Q

    pltpu.sync_copy(c1_hbm.at[sample], c1v)
    pltpu.sync_copy(lab_hbm.at[sample], lv)
    pltpu.sync_copy(ploc_hbm.at[sample], pv)
    pltpu.sync_copy(lloc_hbm.at[sample], llv)
    pltpu.sync_copy(shp_hbm, shv)

    ones = jnp.full((16,), 1.0, jnp.float32)
    zeros = jnp.zeros((16,), jnp.float32)
    w1 = jnp.full((16,), 1.0 - cid.astype(jnp.float32), jnp.float32)
    w2 = ones - w1

    pacc = zeros
    for c in range(NCHUNK):
        ds = pl.ds(c * 16, 16)
        cls1 = c1v[ds]
        labf = lv[ds]
        mf = jnp.where(labf > 0.5, ones, zeros)
        pp = jnp.exp(cls1)

        sh0 = shv[0, ds]
        sh1 = shv[1, ds]
        sh2 = shv[2, ds]
        sh3 = shv[3, ds]

        def corners(locv):
            cx = locv[0, ds] * sh2 + sh0
            cy = locv[1, ds] * sh3 + sh1
            w = jnp.exp(locv[2, ds]) * sh2
            h = jnp.exp(locv[3, ds]) * sh3
            hw = w * 0.5
            hh = h * 0.5
            return cx - hw, cy - hh, cx + hw, cy + hh

        ax1, ay1, ax2, ay2 = corners(pv)
        bx1, by1, bx2, by2 = corners(llv)

        ix1 = jnp.maximum(ax1, bx1)
        iy1 = jnp.maximum(ay1, by1)
        ix2 = jnp.minimum(ax2, bx2)
        iy2 = jnp.minimum(ay2, by2)
        inter = jnp.maximum(ix2 - ix1, 0.0) * jnp.maximum(iy2 - iy1, 0.0)
        area_a = jnp.maximum(ax2 - ax1, 0.0) * jnp.maximum(ay2 - ay1, 0.0)
        area_b = jnp.maximum(bx2 - bx1, 0.0) * jnp.maximum(by2 - by1, 0.0)
        iou = inter / jnp.maximum(area_a + area_b - inter, 1e-6)

        key = w1 * iou + w2 * pp
        val = w1 * pp + w2 * iou
        kv[ds] = key
        uv[ds] = mf * jnp.exp(-GAMMA * val)
        vv[ds] = mf * jnp.exp(GAMMA * val)
        pacc = pacc + mf

    iotav = lax.broadcasted_iota(jnp.int32, (16,), 0)
    gdn = lax.GatherDimensionNumbers(
        offset_dims=(), collapsed_slice_dims=(0,), start_index_map=(0,))

    def lane_bcast(vec, j):
        idx = jnp.full((16,), j, jnp.int32)
        return lax.gather(vec, idx[:, None], gdn, (1,),
                          mode=lax.GatherScatterMode.PROMISE_IN_BOUNDS)

    def body(bc, sacc):
        base = bc * 16
        bds = pl.ds(base, 16)
        kb16 = kv[bds]
        kbs = [lane_bcast(kb16, j) for j in range(16)]
        accs0 = (zeros,) * 16

        def low(c, accs):
            ka = kv[pl.ds(c * 16, 16)]
            ua = uv[pl.ds(c * 16, 16)]
            return tuple(
                accs[j] + jnp.where(ka >= kbs[j], ua, zeros)
                for j in range(16))

        def up(c, accs):
            ka = kv[pl.ds(c * 16, 16)]
            ua = uv[pl.ds(c * 16, 16)]
            return tuple(
                accs[j] + jnp.where(ka > kbs[j], ua, zeros)
                for j in range(16))

        accs = lax.fori_loop(0, bc, low, accs0)
        accs = lax.fori_loop(bc + 1, NCHUNK, up, accs)

        udiag = uv[bds]
        accs = list(accs)
        for j in range(16):
            tie = (kb16 == kbs[j]) & (iotav < j)
            pred = (kb16 > kbs[j]) | tie
            accs[j] = accs[j] + jnp.where(pred, udiag, zeros)

        vbc = vv[bds]
        total = zeros
        for j in range(16):
            total = total + accs[j] * lane_bcast(vbc, j)
        return sacc + total

    sacc = lax.fori_loop(bslice * CPS, (bslice + 1) * CPS, body, zeros)

    sv[0, :] = sacc
    sv[1, :] = pacc
    pltpu.sync_copy(sv, out_hbm.at[sid * 2 + cid])


def _sc_call(x):
    mesh = plsc.VectorSubcoreMesh(core_axis_name="c", subcore_axis_name="s")
    k = functools.partial(
        pl.kernel,
        mesh=mesh,
        out_type=jax.ShapeDtypeStruct((2 * B, 2, 16), jnp.float32),
        scratch_types=[
            pltpu.VMEM((NPAD,), jnp.float32),
            pltpu.VMEM((NPAD,), jnp.float32),
            pltpu.VMEM((4, NPAD), jnp.float32),
            pltpu.VMEM((4, NPAD), jnp.float32),
            pltpu.VMEM((4, NPAD), jnp.float32),
            pltpu.VMEM((NPAD,), jnp.float32),
            pltpu.VMEM((NPAD,), jnp.float32),
            pltpu.VMEM((NPAD,), jnp.float32),
            pltpu.VMEM((2, 16), jnp.float32),
        ],
    )(_sc_task_body)
    return k(*x)


def _tc_main_kernel(cls1_ref, lab_ref, ploc_ref, lloc_ref, shp_ref,
                    t1_ref, t2_ref, cnt_ref):
    b = pl.program_id(0)

    @pl.when(b == 0)
    def _init():
        t1_ref[...] = jnp.zeros((1, 1), jnp.float32)
        t2_ref[...] = jnp.zeros((1, 1), jnp.float32)
        cnt_ref[...] = jnp.zeros((1, 1), jnp.float32)

    lab = lab_ref[0]
    m = lab > 0.5
    mf = m.astype(jnp.float32)
    pos_prob = jnp.exp(cls1_ref[0])

    ploc = ploc_ref[0]
    lloc = lloc_ref[0]
    shp = shp_ref[...]
    sh0, sh1 = shp[0:1, :], shp[1:2, :]
    sh2, sh3 = shp[2:3, :], shp[3:4, :]

    def corners(loc):
        cx = loc[0:1, :] * sh2 + sh0
        cy = loc[1:2, :] * sh3 + sh1
        w = jnp.exp(loc[2:3, :]) * sh2
        h = jnp.exp(loc[3:4, :]) * sh3
        return cx - w * 0.5, cy - h * 0.5, cx + w * 0.5, cy + h * 0.5

    ax1, ay1, ax2, ay2 = corners(ploc)
    bx1, by1, bx2, by2 = corners(lloc)
    ix1 = jnp.maximum(ax1, bx1)
    iy1 = jnp.maximum(ay1, by1)
    ix2 = jnp.minimum(ax2, bx2)
    iy2 = jnp.minimum(ay2, by2)
    inter = jnp.maximum(ix2 - ix1, 0.0) * jnp.maximum(iy2 - iy1, 0.0)
    area_a = jnp.maximum(ax2 - ax1, 0.0) * jnp.maximum(ay2 - ay1, 0.0)
    area_b = jnp.maximum(bx2 - bx1, 0.0) * jnp.maximum(by2 - by1, 0.0)
    iou = inter / jnp.maximum(area_a + area_b - inter, 1e-6)

    p_count = jnp.sum(mf)
    npairs = jnp.maximum(p_count * (p_count - 1.0) * 0.5, 1.0)
    include = (p_count >= 2.0).astype(jnp.float32)
    scale = include / npairs

    row = jax.lax.broadcasted_iota(jnp.int32, (NPAD, NPAD), 0)
    col = jax.lax.broadcasted_iota(jnp.int32, (NPAD, NPAD), 1)
    tie = row < col

    def pair_sum(key, val):
        u = mf * jnp.exp(-GAMMA * val)
        v = mf * jnp.exp(GAMMA * val)
        kc = key.reshape(NPAD, 1)
        before = (kc > key) | ((kc == key) & tie)
        prod = u.reshape(NPAD, 1) * v
        return jnp.sum(jnp.where(before, prod, 0.0))

    s1 = pair_sum(iou, pos_prob)
    s2 = pair_sum(pos_prob, iou)

    t1_ref[...] += (s1 * scale).reshape(1, 1)
    t2_ref[...] += (s2 * scale).reshape(1, 1)
    cnt_ref[...] += include.reshape(1, 1)


def _finalize_kernel(x_ref, t1_ref, t2_ref, cnt_ref, f1_ref, f2_ref):
    x = x_ref[...]
    s = jnp.sum(x[:, 0, :], axis=1, keepdims=True)
    p = jnp.sum(x[:, 1, :], axis=1, keepdims=True)
    rowid = lax.broadcasted_iota(jnp.int32, (2 * B, 1), 0)
    is1 = (rowid % 2) == 0
    npairs = jnp.maximum(p * (p - 1.0) * 0.5, 1.0)
    include = (p >= 2.0).astype(jnp.float32)
    contrib = include * s / npairs
    total1 = jnp.sum(jnp.where(is1, contrib, 0.0)).reshape(1, 1) + t1_ref[...]
    total2 = jnp.sum(jnp.where(is1, 0.0, contrib)).reshape(1, 1) + t2_ref[...]
    count = (jnp.sum(jnp.where(is1, include, 0.0)) / Q).reshape(1, 1) \
        + cnt_ref[...]
    denom = jnp.maximum(count, 1.0)
    has = (count > 0.0).astype(jnp.float32)
    f1_ref[...] = total1 / denom * has
    f2_ref[...] = total2 / denom * has


def kernel(cls, label_cls, pred_loc, label_loc, shape):
    pad = NPAD - N
    cls1 = jnp.pad(cls.reshape(B, N, 2)[:, :, 1], ((0, 0), (0, pad)))
    labf = jnp.pad(label_cls.reshape(B, N).astype(jnp.float32),
                   ((0, 0), (0, pad)))
    ploc = jnp.pad(pred_loc.reshape(B, 4, N), ((0, 0), (0, 0), (0, pad)))
    lloc = jnp.pad(label_loc.reshape(B, 4, N), ((0, 0), (0, 0), (0, pad)))
    shp = jnp.pad(shape.reshape(4, N), ((0, 0), (0, pad)),
                  constant_values=1.0)

    parts = _sc_call((cls1[:NSC], labf[:NSC], ploc[:NSC], lloc[:NSC], shp))

    cls1_t = cls1[NSC:].reshape(NTC, 1, NPAD)
    labf_t = labf[NSC:].reshape(NTC, 1, NPAD)
    t1, t2, cnt = pl.pallas_call(
        _tc_main_kernel,
        grid=(NTC,),
        in_specs=[
            pl.BlockSpec((1, 1, NPAD), lambda b: (b, 0, 0)),
            pl.BlockSpec((1, 1, NPAD), lambda b: (b, 0, 0)),
            pl.BlockSpec((1, 4, NPAD), lambda b: (b, 0, 0)),
            pl.BlockSpec((1, 4, NPAD), lambda b: (b, 0, 0)),
            pl.BlockSpec((4, NPAD), lambda b: (0, 0)),
        ],
        out_specs=[
            pl.BlockSpec((1, 1), lambda b: (0, 0)),
            pl.BlockSpec((1, 1), lambda b: (0, 0)),
            pl.BlockSpec((1, 1), lambda b: (0, 0)),
        ],
        out_shape=[
            jax.ShapeDtypeStruct((1, 1), jnp.float32),
            jax.ShapeDtypeStruct((1, 1), jnp.float32),
            jax.ShapeDtypeStruct((1, 1), jnp.float32),
        ],
    )(cls1_t, labf_t, ploc[NSC:], lloc[NSC:], shp)

    f1, f2 = pl.pallas_call(
        _finalize_kernel,
        out_shape=[
            jax.ShapeDtypeStruct((1, 1), jnp.float32),
            jax.ShapeDtypeStruct((1, 1), jnp.float32),
        ],
    )(parts, t1, t2, cnt)
    return (f1.reshape(()), f2.reshape(()))
